# baseline (speedup 1.0000x reference)
import functools

import jax
import jax.numpy as jnp
from jax import lax
from jax.experimental import pallas as pl
from jax.experimental.pallas import tpu as pltpu
from jax.sharding import Mesh, PartitionSpec as P

try:
    from jax import shard_map as _shard_map_fn

    def _shard_map(f, mesh, in_specs, out_specs):
        return _shard_map_fn(f, mesh=mesh, in_specs=in_specs,
                             out_specs=out_specs, check_vma=False)
except ImportError:
    from jax.experimental.shard_map import shard_map as _shard_map_fn

    def _shard_map(f, mesh, in_specs, out_specs):
        return _shard_map_fn(f, mesh=mesh, in_specs=in_specs,
                             out_specs=out_specs, check_rep=False)


def _bn_prep(w, g, rs, gam, bet, b, nr, m_count, eps, row, col):
    t = jnp.dot(w, g, preferred_element_type=jnp.float32)
    s2raw = jnp.sum(t * w, axis=1, keepdims=True)
    sraw = jnp.dot(w, rs, preferred_element_type=jnp.float32)
    pool = jnp.where((row // nr) == (col // nr), 1.0, 0.0)
    s_p = jnp.dot(pool, sraw, preferred_element_type=jnp.float32)
    s2_p = jnp.dot(pool, s2raw, preferred_element_type=jnp.float32)
    s = s_p + m_count * b
    s2 = s2_p + 2.0 * b * s_p + m_count * b * b
    mean = s / m_count
    var = jnp.maximum(s2 / m_count - mean * mean, 0.0)
    scale = gam * lax.rsqrt(var + eps)
    shift = scale * (b - mean) + bet
    return w * scale, shift


def _circulant_w(tap_ref, k_dim, nr):
    row = lax.broadcasted_iota(jnp.int32, (k_dim, k_dim), 0)
    col = lax.broadcasted_iota(jnp.int32, (k_dim, k_dim), 1)
    diff = (col - row) & (nr - 1)
    w = jnp.where(diff == nr - 1, tap_ref[0], 0.0)
    w = w + jnp.where(diff == 0, tap_ref[1], 0.0)
    w = w + jnp.where(diff == 1, tap_ref[2], 0.0)
    return w, row, col


def _fused2_kernel(x_ref, tap_ref, gam_ref, bet_ref, b_ref, o_ref,
                   xs_ref, g_ref, rs_ref, w2_ref, sh_ref,
                   *, nr, bb, hb, m_count, eps, slope):
    s = pl.program_id(0)
    k_dim = g_ref.shape[0]
    p0 = 2 * hb

    @pl.when(s == 0)
    def _init():
        g_ref[...] = jnp.zeros_like(g_ref)
        rs_ref[...] = jnp.zeros_like(rs_ref)

    @pl.when(s < p0)
    def _stats():
        for i in range(bb):
            xb = x_ref[i]
            g_ref[...] += lax.dot_general(xb, xb, (((1,), (1,)), ((), ())),
                                          preferred_element_type=jnp.float32)
            rs_ref[...] += jnp.sum(xb, axis=1, keepdims=True)

    @pl.when((s >= hb) & (s < p0))
    def _stash():
        for i in range(bb):
            xs_ref[(s - hb) * bb + i] = x_ref[i].astype(jnp.bfloat16)

    @pl.when(s == p0)
    def _prep():
        w, row, col = _circulant_w(tap_ref, k_dim, nr)
        w2_ref[...], sh_ref[...] = _bn_prep(
            w, g_ref[...], rs_ref[...], gam_ref[...], bet_ref[...],
            b_ref[...], nr, m_count, eps, row, col)

    @pl.when(s >= p0)
    def _apply():
        for i in range(bb):
            xb16 = xs_ref[(s - p0) * bb + i]
            y = jnp.dot(w2_ref[...], xb16, preferred_element_type=jnp.float32)
            y = y + sh_ref[...]
            o_ref[i] = jnp.maximum(y, slope * y).astype(o_ref.dtype)


def _fused1_kernel(x_ref, tap_ref, gam_ref, bet_ref, b_ref, o_ref,
                   xs_ref, g_ref, rs_ref, w2_ref, sh_ref,
                   *, nr, bb, m_count, eps, slope):
    ph = pl.program_id(0)
    j = pl.program_id(1)
    k_dim = g_ref.shape[0]

    @pl.when((ph == 0) & (j == 0))
    def _init():
        g_ref[...] = jnp.zeros_like(g_ref)
        rs_ref[...] = jnp.zeros_like(rs_ref)

    @pl.when(ph == 0)
    def _stats():
        for i in range(bb):
            xb = x_ref[i]
            g_ref[...] += lax.dot_general(xb, xb, (((1,), (1,)), ((), ())),
                                          preferred_element_type=jnp.float32)
            rs_ref[...] += jnp.sum(xb, axis=1, keepdims=True)
            xs_ref[bb * j + i] = xb.astype(jnp.bfloat16)

    @pl.when((ph == 1) & (j == 0))
    def _prep():
        w, row, col = _circulant_w(tap_ref, k_dim, nr)
        w2_ref[...], sh_ref[...] = _bn_prep(
            w, g_ref[...], rs_ref[...], gam_ref[...], bet_ref[...],
            b_ref[...], nr, m_count, eps, row, col)

    @pl.when(ph == 1)
    def _apply():
        for i in range(bb):
            xb16 = xs_ref[bb * j + i]
            y = jnp.dot(w2_ref[...], xb16, preferred_element_type=jnp.float32)
            y = y + sh_ref[...]
            o_ref[i] = jnp.maximum(y, slope * y).astype(o_ref.dtype)


def kernel(x, conv_w, conv_b, bn_gamma, bn_beta, *, eps=1e-5, slope=0.1):
    B, C, Nr, Np = x.shape
    K = C * Nr
    M = B * Np * Nr
    f32 = jnp.float32
    assert Nr & (Nr - 1) == 0, "ring dim assumed power of two"

    xf = x.reshape(B, K, Np)
    tap = jnp.broadcast_to(
        conv_w.astype(f32).transpose(2, 0, 1)[:, :, None, :, None],
        (3, C, Nr, C, Nr)).reshape(3, K, K)
    b_col = jnp.repeat(conv_b.astype(f32), Nr).reshape(K, 1)
    gam_col = jnp.repeat(bn_gamma.astype(f32), Nr).reshape(K, 1)
    bet_col = jnp.repeat(bn_beta.astype(f32), Nr).reshape(K, 1)

    BB = 4
    const2 = lambda *a: (0, 0)
    const3 = lambda *a: (0, 0, 0)
    devs = jax.devices()

    if len(devs) >= 2 and (B // 2) % BB == 0 and B % 2 == 0:
        HB = (B // 2) // BB
        P0 = 2 * HB

        def x_idx(s):
            d = lax.axis_index("d")
            own = d * HB
            other = (1 - d) * HB
            blk = jnp.where(s < HB, other + s,
                            jnp.where(s < P0, own + (s - HB),
                                      own + HB - 1))
            return (blk, 0, 0)

        def shard_body(xf_, tap_, gam_, bet_, b_):
            return pl.pallas_call(
                functools.partial(_fused2_kernel, nr=Nr, bb=BB, hb=HB,
                                  m_count=float(M), eps=eps, slope=slope),
                grid=(3 * HB,),
                in_specs=[pl.BlockSpec((BB, K, Np), x_idx),
                          pl.BlockSpec((3, K, K), const3),
                          pl.BlockSpec((K, 1), const2),
                          pl.BlockSpec((K, 1), const2),
                          pl.BlockSpec((K, 1), const2)],
                out_specs=pl.BlockSpec(
                    (BB, K, Np),
                    lambda s: (jnp.where(s >= P0, s - P0, 0), 0, 0)),
                out_shape=jax.ShapeDtypeStruct((B // 2, K, Np), x.dtype),
                scratch_shapes=[pltpu.VMEM((B // 2, K, Np), jnp.bfloat16),
                                pltpu.VMEM((K, K), f32),
                                pltpu.VMEM((K, 1), f32),
                                pltpu.VMEM((K, K), f32),
                                pltpu.VMEM((K, 1), f32)],
                compiler_params=pltpu.CompilerParams(
                    dimension_semantics=("arbitrary",),
                    vmem_limit_bytes=48 << 20),
            )(xf_, tap_, gam_, bet_, b_)

        mesh = Mesh(devs[:2], ("d",))
        out_flat = _shard_map(
            shard_body, mesh,
            in_specs=(P(), P(), P(), P(), P()),
            out_specs=P("d"))(xf, tap, gam_col, bet_col, b_col)
        return out_flat.reshape(B, C, Nr, Np)

    J = B // BB
    out_flat = pl.pallas_call(
        functools.partial(_fused1_kernel, nr=Nr, bb=BB, m_count=float(M),
                          eps=eps, slope=slope),
        grid=(2, J),
        in_specs=[pl.BlockSpec((BB, K, Np),
                               lambda ph, j: (jnp.where(ph == 0, j, J - 1),
                                              0, 0)),
                  pl.BlockSpec((3, K, K), const3),
                  pl.BlockSpec((K, 1), const2),
                  pl.BlockSpec((K, 1), const2),
                  pl.BlockSpec((K, 1), const2)],
        out_specs=pl.BlockSpec((BB, K, Np),
                               lambda ph, j: (jnp.where(ph == 1, j, 0),
                                              0, 0)),
        out_shape=jax.ShapeDtypeStruct((B, K, Np), x.dtype),
        scratch_shapes=[pltpu.VMEM((B, K, Np), jnp.bfloat16),
                        pltpu.VMEM((K, K), f32),
                        pltpu.VMEM((K, 1), f32),
                        pltpu.VMEM((K, K), f32),
                        pltpu.VMEM((K, 1), f32)],
        compiler_params=pltpu.CompilerParams(
            dimension_semantics=("arbitrary", "arbitrary"),
            vmem_limit_bytes=56 << 20),
    )(xf, tap, gam_col, bet_col, b_col)
    return out_flat.reshape(B, C, Nr, Np)


# speedup vs baseline: 4.3202x; 4.3202x over previous
import functools

import jax
import jax.numpy as jnp
from jax import lax
from jax.experimental import pallas as pl
from jax.experimental.pallas import tpu as pltpu


def _bn_prep(w, g, rs, gam, bet, b, nr, m_count, eps, row, col):
    t = jnp.dot(w, g, preferred_element_type=jnp.float32)
    s2raw = jnp.sum(t * w, axis=1, keepdims=True)
    sraw = jnp.dot(w, rs, preferred_element_type=jnp.float32)
    pool = jnp.where((row // nr) == (col // nr), 1.0, 0.0)
    s_p = jnp.dot(pool, sraw, preferred_element_type=jnp.float32)
    s2_p = jnp.dot(pool, s2raw, preferred_element_type=jnp.float32)
    s = s_p + m_count * b
    s2 = s2_p + 2.0 * b * s_p + m_count * b * b
    mean = s / m_count
    var = jnp.maximum(s2 / m_count - mean * mean, 0.0)
    scale = gam * lax.rsqrt(var + eps)
    shift = scale * (b - mean) + bet
    return w * scale, shift


def _circulant_w(tap_ref, k_dim, nr):
    row = lax.broadcasted_iota(jnp.int32, (k_dim, k_dim), 0)
    col = lax.broadcasted_iota(jnp.int32, (k_dim, k_dim), 1)
    diff = (col - row) & (nr - 1)
    w = jnp.where(diff == nr - 1, tap_ref[0], 0.0)
    w = w + jnp.where(diff == 0, tap_ref[1], 0.0)
    w = w + jnp.where(diff == 1, tap_ref[2], 0.0)
    return w, row, col


def _fused_kernel(x_ref, tap_ref, gam_ref, bet_ref, b_ref, o_ref,
                  xs_ref, g_ref, rs_ref, w2_ref, sh_ref,
                  *, nr, bi, bo, ns, m_count, eps, slope):
    s = pl.program_id(0)
    k_dim = g_ref.shape[0]

    @pl.when(s == 0)
    def _init():
        g_ref[...] = jnp.zeros_like(g_ref)
        rs_ref[...] = jnp.zeros_like(rs_ref)

    @pl.when(s < ns)
    def _stats():
        for i in range(bi):
            xb = x_ref[i]
            g_ref[...] += lax.dot_general(xb, xb, (((1,), (1,)), ((), ())),
                                          preferred_element_type=jnp.float32)
            rs_ref[...] += jnp.sum(xb, axis=1, keepdims=True)
            xs_ref[bi * s + i] = xb.astype(jnp.bfloat16)

    @pl.when(s == ns)
    def _prep():
        w, row, col = _circulant_w(tap_ref, k_dim, nr)
        w2_ref[...], sh_ref[...] = _bn_prep(
            w, g_ref[...], rs_ref[...], gam_ref[...], bet_ref[...],
            b_ref[...], nr, m_count, eps, row, col)

    @pl.when(s >= ns)
    def _apply():
        for i in range(bo):
            xb16 = xs_ref[bo * (s - ns) + i]
            y = jnp.dot(w2_ref[...], xb16, preferred_element_type=jnp.float32)
            y = y + sh_ref[...]
            o_ref[i] = jnp.maximum(y, slope * y).astype(o_ref.dtype)


def kernel(x, conv_w, conv_b, bn_gamma, bn_beta, *, eps=1e-5, slope=0.1):
    B, C, Nr, Np = x.shape
    K = C * Nr
    M = B * Np * Nr
    f32 = jnp.float32
    assert Nr & (Nr - 1) == 0, "ring dim assumed power of two"

    xf = x.reshape(B, K, Np)
    tap = jnp.broadcast_to(
        conv_w.astype(f32).transpose(2, 0, 1)[:, :, None, :, None],
        (3, C, Nr, C, Nr)).reshape(3, K, K)
    b_col = jnp.repeat(conv_b.astype(f32), Nr).reshape(K, 1)
    gam_col = jnp.repeat(bn_gamma.astype(f32), Nr).reshape(K, 1)
    bet_col = jnp.repeat(bn_beta.astype(f32), Nr).reshape(K, 1)

    BI = 8
    BO = 4
    NS = B // BI
    NA = B // BO
    x_spec = pl.BlockSpec((BI, K, Np),
                          lambda s: (jnp.where(s < NS, s, NS - 1), 0, 0))
    o_spec = pl.BlockSpec((BO, K, Np),
                          lambda s: (jnp.where(s >= NS, s - NS, 0), 0, 0))
    const2 = lambda s: (0, 0)
    const3 = lambda s: (0, 0, 0)

    out_flat = pl.pallas_call(
        functools.partial(_fused_kernel, nr=Nr, bi=BI, bo=BO, ns=NS,
                          m_count=float(M), eps=eps, slope=slope),
        grid=(NS + NA,),
        in_specs=[x_spec,
                  pl.BlockSpec((3, K, K), const3),
                  pl.BlockSpec((K, 1), const2),
                  pl.BlockSpec((K, 1), const2),
                  pl.BlockSpec((K, 1), const2)],
        out_specs=o_spec,
        out_shape=jax.ShapeDtypeStruct((B, K, Np), x.dtype),
        scratch_shapes=[pltpu.VMEM((B, K, Np), jnp.bfloat16),
                        pltpu.VMEM((K, K), f32),
                        pltpu.VMEM((K, 1), f32),
                        pltpu.VMEM((K, K), f32),
                        pltpu.VMEM((K, 1), f32)],
        compiler_params=pltpu.CompilerParams(
            dimension_semantics=("arbitrary",),
            vmem_limit_bytes=61 << 20),
    )(xf, tap, gam_col, bet_col, b_col)
    return out_flat.reshape(B, C, Nr, Np)


# speedup vs baseline: 4.3367x; 1.0038x over previous
import functools

import jax
import jax.numpy as jnp
from jax import lax
from jax.experimental import pallas as pl
from jax.experimental.pallas import tpu as pltpu


def _bn_prep(w, g, rs, gam, bet, b, nr, m_count, eps, row, col):
    t = jnp.dot(w, g, preferred_element_type=jnp.float32)
    s2raw = jnp.sum(t * w, axis=1, keepdims=True)
    sraw = jnp.dot(w, rs, preferred_element_type=jnp.float32)
    pool = jnp.where((row // nr) == (col // nr), 1.0, 0.0)
    s_p = jnp.dot(pool, sraw, preferred_element_type=jnp.float32)
    s2_p = jnp.dot(pool, s2raw, preferred_element_type=jnp.float32)
    s = s_p + m_count * b
    s2 = s2_p + 2.0 * b * s_p + m_count * b * b
    mean = s / m_count
    var = jnp.maximum(s2 / m_count - mean * mean, 0.0)
    scale = gam * lax.rsqrt(var + eps)
    shift = scale * (b - mean) + bet
    return w * scale, shift


def _circulant_w(tap_ref, k_dim, nr):
    row = lax.broadcasted_iota(jnp.int32, (k_dim, k_dim), 0)
    col = lax.broadcasted_iota(jnp.int32, (k_dim, k_dim), 1)
    diff = (col - row) & (nr - 1)
    w = jnp.where(diff == nr - 1, tap_ref[0], 0.0)
    w = w + jnp.where(diff == 0, tap_ref[1], 0.0)
    w = w + jnp.where(diff == 1, tap_ref[2], 0.0)
    return w, row, col


def _fused_kernel(x_ref, tap_ref, gam_ref, bet_ref, b_ref, o_ref,
                  xs_ref, g_ref, rs_ref, w2_ref, sh_ref,
                  *, nr, bi, bo, ns, m_count, eps, slope):
    s = pl.program_id(0)
    k_dim = g_ref.shape[0]

    @pl.when(s == 0)
    def _init():
        g_ref[...] = jnp.zeros_like(g_ref)
        rs_ref[...] = jnp.zeros_like(rs_ref)

    @pl.when(s < ns)
    def _stats():
        for i in range(bi):
            xb = x_ref[i]
            g_ref[...] += lax.dot_general(xb, xb, (((1,), (1,)), ((), ())),
                                          preferred_element_type=jnp.float32)
            rs_ref[...] += jnp.sum(xb, axis=1, keepdims=True)
            xs_ref[bi * s + i] = xb.astype(jnp.bfloat16)

    @pl.when(s == ns)
    def _prep():
        w, row, col = _circulant_w(tap_ref, k_dim, nr)
        w2_ref[...], sh_ref[...] = _bn_prep(
            w, g_ref[...], rs_ref[...], gam_ref[...], bet_ref[...],
            b_ref[...], nr, m_count, eps, row, col)

    @pl.when(s >= ns)
    def _apply():
        for i in range(bo):
            xb16 = xs_ref[bo * (s - ns) + i]
            y = jnp.dot(w2_ref[...], xb16, preferred_element_type=jnp.float32)
            y = y + sh_ref[...]
            o_ref[i] = jnp.maximum(y, slope * y).astype(o_ref.dtype)


def kernel(x, conv_w, conv_b, bn_gamma, bn_beta, *, eps=1e-5, slope=0.1):
    B, C, Nr, Np = x.shape
    K = C * Nr
    M = B * Np * Nr
    f32 = jnp.float32
    assert Nr & (Nr - 1) == 0, "ring dim assumed power of two"

    xf = x.reshape(B, K, Np)
    tap = jnp.broadcast_to(
        conv_w.astype(f32).transpose(2, 0, 1)[:, :, None, :, None],
        (3, C, Nr, C, Nr)).reshape(3, K, K)
    b_col = jnp.repeat(conv_b.astype(f32), Nr).reshape(K, 1)
    gam_col = jnp.repeat(bn_gamma.astype(f32), Nr).reshape(K, 1)
    bet_col = jnp.repeat(bn_beta.astype(f32), Nr).reshape(K, 1)

    BI = next(b for b in (8, 4, 2, 1) if B % b == 0)
    BO = next(b for b in (4, 2, 1) if B % b == 0)
    NS = B // BI
    NA = B // BO
    x_spec = pl.BlockSpec((BI, K, Np),
                          lambda s: (jnp.where(s < NS, s, NS - 1), 0, 0))
    o_spec = pl.BlockSpec((BO, K, Np),
                          lambda s: (jnp.where(s >= NS, s - NS, 0), 0, 0))
    const2 = lambda s: (0, 0)
    const3 = lambda s: (0, 0, 0)

    out_flat = pl.pallas_call(
        functools.partial(_fused_kernel, nr=Nr, bi=BI, bo=BO, ns=NS,
                          m_count=float(M), eps=eps, slope=slope),
        grid=(NS + NA,),
        in_specs=[x_spec,
                  pl.BlockSpec((3, K, K), const3),
                  pl.BlockSpec((K, 1), const2),
                  pl.BlockSpec((K, 1), const2),
                  pl.BlockSpec((K, 1), const2)],
        out_specs=o_spec,
        out_shape=jax.ShapeDtypeStruct((B, K, Np), x.dtype),
        scratch_shapes=[pltpu.VMEM((B, K, Np), jnp.bfloat16),
                        pltpu.VMEM((K, K), f32),
                        pltpu.VMEM((K, 1), f32),
                        pltpu.VMEM((K, K), f32),
                        pltpu.VMEM((K, 1), f32)],
        compiler_params=pltpu.CompilerParams(
            dimension_semantics=("arbitrary",),
            vmem_limit_bytes=61 << 20),
    )(xf, tap, gam_col, bet_col, b_col)
    return out_flat.reshape(B, C, Nr, Np)
